# baseline (speedup 1.0000x reference)
import functools

import jax
import jax.numpy as jnp
from jax import lax
from jax.experimental import pallas as pl
from jax.experimental.pallas import tpu as pltpu
from jax.experimental.pallas import tpu_sc as plsc

D = 64
B = 16384
K = 10
L = 16
NC, NS = 2, 16
NW = NC * NS
BPW = B // NW
G = 16
NG = BPW // G
NBUF = 4


def _sc_scores(center, context, ns_flat, w_center, w_context):
    mesh = plsc.VectorSubcoreMesh(core_axis_name="c", subcore_axis_name="s")

    @functools.partial(
        pl.kernel,
        out_type=[
            jax.ShapeDtypeStruct((NW, BPW), jnp.float32),
            jax.ShapeDtypeStruct((NW, K, BPW), jnp.float32),
        ],
        mesh=mesh,
        compiler_params=pltpu.CompilerParams(
            needs_layout_passes=False, use_tc_tiling_on_sc=True),
        scratch_types=[
            pltpu.VMEM((BPW,), jnp.int32),
            pltpu.VMEM((BPW,), jnp.int32),
            pltpu.VMEM((BPW * K,), jnp.int32),
            pltpu.VMEM((NBUF, G, D), jnp.float32),
            pltpu.VMEM((NBUF, G, D), jnp.float32),
            pltpu.VMEM((NBUF, G * K, D), jnp.float32),
            pltpu.VMEM((BPW,), jnp.float32),
            pltpu.VMEM((K, BPW), jnp.float32),
            pltpu.SemaphoreType.DMA,
            pltpu.SemaphoreType.DMA,
            pltpu.SemaphoreType.DMA,
            pltpu.SemaphoreType.DMA,
        ],
    )
    def sc_kernel(center_hbm, context_hbm, ns_hbm, wc_hbm, wx_hbm,
                  pos_out, neg_out,
                  cidx, xidx, nidx, cbuf, xbuf, nbuf, posv, negv,
                  sem0, sem1, sem2, sem3):
        wid = lax.axis_index("s") * NC + lax.axis_index("c")
        base = wid * BPW
        sems = [sem0, sem1, sem2, sem3]

        pltpu.sync_copy(center_hbm.at[pl.ds(base, BPW)], cidx)
        pltpu.sync_copy(context_hbm.at[pl.ds(base, BPW)], xidx)
        pltpu.sync_copy(ns_hbm.at[pl.ds(base * K, BPW * K)], nidx)

        iota = lax.iota(jnp.int32, L)

        def fire(g, s, sem):
            off = g * G

            def cx_body(i, carry):
                pos = jnp.full((L,), off, jnp.int32) + i
                crow = plsc.load_gather(cidx, [pos])[0]
                xrow = plsc.load_gather(xidx, [pos])[0]
                pltpu.async_copy(wc_hbm.at[crow], cbuf.at[s, i], sem)
                pltpu.async_copy(wx_hbm.at[xrow], xbuf.at[s, i], sem)
                return carry

            lax.fori_loop(0, G, cx_body, jnp.int32(0))

            def ns_body(r, carry):
                pos = jnp.full((L,), off * K, jnp.int32) + r
                nrow_s = plsc.load_gather(nidx, [pos])[0]
                pltpu.async_copy(wx_hbm.at[nrow_s], nbuf.at[s, r], sem)
                return carry

            lax.fori_loop(0, G * K, ns_body, jnp.int32(0))

        def drain(s, sem):
            pltpu.make_async_copy(wc_hbm.at[pl.ds(0, G)], cbuf.at[s], sem).wait()
            pltpu.make_async_copy(wc_hbm.at[pl.ds(0, G)], xbuf.at[s], sem).wait()
            pltpu.make_async_copy(wc_hbm.at[pl.ds(0, G * K)], nbuf.at[s], sem).wait()

        def compute(g, s):
            off = g * G
            nrow = [iota * K + j for j in range(K)]

            def dbody(d, accs):
                rot = (jnp.full((L,), 0, jnp.int32) + d + iota) & (D - 1)
                ca = plsc.load_gather(cbuf.at[s], [iota, rot])
                xa = plsc.load_gather(xbuf.at[s], [iota, rot])
                out = [accs[0] + ca * xa]
                for j in range(K):
                    na = plsc.load_gather(nbuf.at[s], [nrow[j], rot])
                    out.append(accs[1 + j] + ca * na)
                return tuple(out)

            zero = jnp.zeros((L,), jnp.float32)
            accs = lax.fori_loop(0, D, dbody, tuple(zero for _ in range(K + 1)))
            posv[pl.ds(off, G)] = accs[0]
            for j in range(K):
                negv[j, pl.ds(off, G)] = accs[1 + j]

        for s in range(NBUF):
            fire(jnp.int32(s), s, sems[s])

        def outer(i, carry):
            for s in range(NBUF):
                g = i * NBUF + s
                drain(s, sems[s])
                compute(g, s)

                @pl.when(g + NBUF < NG)
                def _():
                    fire(g + NBUF, s, sems[s])
            return carry

        lax.fori_loop(0, NG // NBUF, outer, jnp.int32(0))

        pltpu.sync_copy(posv, pos_out.at[wid])
        pltpu.sync_copy(negv, neg_out.at[wid])

    return sc_kernel(center, context, ns_flat, w_center, w_context)


def _loss_body(p_ref, n_ref, o_ref):
    s_pos = jnp.sum(jax.nn.log_sigmoid(p_ref[...]))
    s_neg = jnp.sum(jax.nn.log_sigmoid(-n_ref[...]))
    o_ref[0, 0] = -(s_pos + s_neg)


def _tc_loss(pos2d, neg2d):
    return pl.pallas_call(
        _loss_body,
        out_shape=jax.ShapeDtypeStruct((1, 1), jnp.float32),
        out_specs=pl.BlockSpec(memory_space=pltpu.SMEM),
    )(pos2d, neg2d)


def kernel(center, context, ns, W_center, W_context):
    center = center.astype(jnp.int32)
    context = context.astype(jnp.int32)
    ns_flat = ns.reshape(-1).astype(jnp.int32)
    pos, neg = _sc_scores(center, context, ns_flat, W_center, W_context)
    loss = _tc_loss(pos.reshape(B // 128, 128), neg.reshape(B * K // 128, 128))
    return loss[0, 0]


# speedup vs baseline: 1.0542x; 1.0542x over previous
import functools

import jax
import jax.numpy as jnp
from jax import lax
from jax.experimental import pallas as pl
from jax.experimental.pallas import tpu as pltpu
from jax.experimental.pallas import tpu_sc as plsc

D = 64
B = 16384
K = 10
L = 16
NC, NS = 2, 16
NW = NC * NS
BPW = B // NW
G = 16
NG = BPW // G
NBUF = 4


def _sc_scores(center, context, ns_flat, w_center, w_context):
    mesh = plsc.VectorSubcoreMesh(core_axis_name="c", subcore_axis_name="s")

    @functools.partial(
        pl.kernel,
        out_type=[
            jax.ShapeDtypeStruct((NW, BPW), jnp.float32),
            jax.ShapeDtypeStruct((NW, K, BPW), jnp.float32),
        ],
        mesh=mesh,
        compiler_params=pltpu.CompilerParams(
            needs_layout_passes=False, use_tc_tiling_on_sc=True),
        scratch_types=[
            pltpu.VMEM((BPW,), jnp.int32),
            pltpu.VMEM((BPW,), jnp.int32),
            pltpu.VMEM((BPW * K,), jnp.int32),
            pltpu.VMEM((NBUF, G, D), jnp.float32),
            pltpu.VMEM((NBUF, G, D), jnp.float32),
            pltpu.VMEM((NBUF, G * K, D), jnp.float32),
            pltpu.VMEM((BPW,), jnp.float32),
            pltpu.VMEM((K, BPW), jnp.float32),
            pltpu.SemaphoreType.DMA,
            pltpu.SemaphoreType.DMA,
            pltpu.SemaphoreType.DMA,
            pltpu.SemaphoreType.DMA,
        ],
    )
    def sc_kernel(center_hbm, context_hbm, ns_hbm, wc_hbm, wx_hbm,
                  pos_out, neg_out,
                  cidx, xidx, nidx, cbuf, xbuf, nbuf, posv, negv,
                  sem0, sem1, sem2, sem3):
        wid = lax.axis_index("s") * NC + lax.axis_index("c")
        base = wid * BPW
        sems = [sem0, sem1, sem2, sem3]

        pltpu.sync_copy(center_hbm.at[pl.ds(base, BPW)], cidx)
        pltpu.sync_copy(context_hbm.at[pl.ds(base, BPW)], xidx)
        pltpu.sync_copy(ns_hbm.at[pl.ds(base * K, BPW * K)], nidx)

        iota = lax.iota(jnp.int32, L)

        def fire(g, s, sem):
            off = g * G
            cvec = cidx[pl.ds(off, G)]
            xvec = xidx[pl.ds(off, G)]
            for i in range(G):
                pltpu.async_copy(wc_hbm.at[cvec[i]], cbuf.at[s, i], sem)
                pltpu.async_copy(wx_hbm.at[xvec[i]], xbuf.at[s, i], sem)
            for j in range(K):
                nvec = nidx[pl.ds(off * K + j * L, L)]
                for i in range(L):
                    pltpu.async_copy(
                        wx_hbm.at[nvec[i]], nbuf.at[s, j * L + i], sem)

        def drain(s, sem):
            pltpu.make_async_copy(wc_hbm.at[pl.ds(0, G)], cbuf.at[s], sem).wait()
            pltpu.make_async_copy(wc_hbm.at[pl.ds(0, G)], xbuf.at[s], sem).wait()
            pltpu.make_async_copy(wc_hbm.at[pl.ds(0, G * K)], nbuf.at[s], sem).wait()

        def compute(g, s):
            off = g * G
            nrow = [iota * K + j for j in range(K)]

            def dbody(d, accs):
                rot = (jnp.full((L,), 0, jnp.int32) + d + iota) & (D - 1)
                ca = plsc.load_gather(cbuf.at[s], [iota, rot])
                xa = plsc.load_gather(xbuf.at[s], [iota, rot])
                out = [accs[0] + ca * xa]
                for j in range(K):
                    na = plsc.load_gather(nbuf.at[s], [nrow[j], rot])
                    out.append(accs[1 + j] + ca * na)
                return tuple(out)

            zero = jnp.zeros((L,), jnp.float32)
            accs = lax.fori_loop(0, D, dbody, tuple(zero for _ in range(K + 1)))
            posv[pl.ds(off, G)] = accs[0]
            for j in range(K):
                negv[j, pl.ds(off, G)] = accs[1 + j]

        for s in range(NBUF):
            fire(jnp.int32(s), s, sems[s])

        def outer(i, carry):
            for s in range(NBUF):
                g = i * NBUF + s
                drain(s, sems[s])
                compute(g, s)

                @pl.when(g + NBUF < NG)
                def _():
                    fire(g + NBUF, s, sems[s])
            return carry

        lax.fori_loop(0, NG // NBUF, outer, jnp.int32(0))

        pltpu.sync_copy(posv, pos_out.at[wid])
        pltpu.sync_copy(negv, neg_out.at[wid])

    return sc_kernel(center, context, ns_flat, w_center, w_context)


def _loss_body(p_ref, n_ref, o_ref):
    s_pos = jnp.sum(jax.nn.log_sigmoid(p_ref[...]))
    s_neg = jnp.sum(jax.nn.log_sigmoid(-n_ref[...]))
    o_ref[0, 0] = -(s_pos + s_neg)


def _tc_loss(pos2d, neg2d):
    return pl.pallas_call(
        _loss_body,
        out_shape=jax.ShapeDtypeStruct((1, 1), jnp.float32),
        out_specs=pl.BlockSpec(memory_space=pltpu.SMEM),
    )(pos2d, neg2d)


def kernel(center, context, ns, W_center, W_context):
    center = center.astype(jnp.int32)
    context = context.astype(jnp.int32)
    ns_flat = ns.reshape(-1).astype(jnp.int32)
    pos, neg = _sc_scores(center, context, ns_flat, W_center, W_context)
    loss = _tc_loss(pos.reshape(B // 128, 128), neg.reshape(B * K // 128, 128))
    return loss[0, 0]
